# baseline (speedup 1.0000x reference)
import functools

import jax
import jax.numpy as jnp
from jax import lax
from jax.experimental import pallas as pl
from jax.experimental.pallas import tpu as pltpu
from jax.experimental.pallas import tpu_sc as plsc

_D = 32
_NU = 1000000
_B = 16384
_L = 16

_info = plsc.get_sparse_core_info()
_NC, _NS = _info.num_cores, _info.num_subcores
_NW = _NC * _NS
_BPW = _B // _NW
_CHUNK = 128
_NCHUNK = _BPW // _CHUNK
_NCOL = (_NU + 127) // 128
_CPW = (_NCOL + _NW - 1) // _NW


def _tp_body(ut_hbm, pt_hbm, ulin_hbm, plin_hbm,
             cb0, cb1, rb0, rb1, si0, si1, so0, so1):
    wid = lax.axis_index("s") * _NC + lax.axis_index("c")
    lanes = lax.iota(jnp.int32, _L)
    lo = wid * _CPW
    hi = jnp.minimum(lo + _CPW, _NCOL)

    for src, dst in ((ut_hbm, ulin_hbm), (pt_hbm, plin_hbm)):
        cbs = (cb0, cb1)
        rbs = (rb0, rb1)
        sis = (si0, si1)
        sos = (so0, so1)

        pltpu.async_copy(src.at[:, pl.ds(lo * 128, 128)], cb0, si0)

        def step(t, carry):
            for k in range(2):
                j = lo + 2 * t + k
                cb, rb, si, so = cbs[k], rbs[k], sis[k], sos[k]

                @pl.when(j < hi)
                def _do():
                    pltpu.make_async_copy(
                        src.at[:, pl.ds(0, 128)], cb, si).wait()

                    @pl.when(j + 1 < hi)
                    def _prefetch():
                        pltpu.async_copy(
                            src.at[:, pl.ds((j + 1) * 128, 128)],
                            cbs[1 - k], sis[1 - k])

                    @pl.when(t >= 1)
                    def _drain():
                        pltpu.make_async_copy(
                            rb, dst.at[pl.ds(0, 4096)], so).wait()

                    for g in range(8):
                        ridx = (g * _L + lanes) * _D
                        for d in range(_D):
                            v = cb[d, pl.ds(g * _L, _L)]
                            plsc.store_scatter(rb, [ridx + d], v)
                    pltpu.async_copy(
                        rb, dst.at[pl.ds(j * 4096, 4096)], so)
            return carry

        lax.fori_loop(0, (_CPW + 1) // 2, step, 0)
        pltpu.make_async_copy(rb0, dst.at[pl.ds(0, 4096)], so0).wait()
        pltpu.make_async_copy(rb1, dst.at[pl.ds(0, 4096)], so1).wait()


def _cf_body(uid_hbm, pid_hbm, ut_hbm, pt_hbm, out_hbm,
             uid_v, pid_v, urows, prows, outc, sem_u, sem_p):
    wid = lax.axis_index("s") * _NC + lax.axis_index("c")
    pltpu.sync_copy(uid_hbm.at[pl.ds(wid * _NCHUNK, _NCHUNK)], uid_v)
    pltpu.sync_copy(pid_hbm.at[pl.ds(wid * _NCHUNK, _NCHUNK)], pid_v)
    copies = []
    for j in range(_NCHUNK):
        copies.append(pltpu.async_copy(
            ut_hbm.at[uid_v.at[j]], urows.at[pl.ds(j * _CHUNK, _CHUNK)], sem_u))
        copies.append(pltpu.async_copy(
            pt_hbm.at[pid_v.at[j]], prows.at[pl.ds(j * _CHUNK, _CHUNK)], sem_p))
    for c in copies:
        c.wait()

    lanes = lax.iota(jnp.int32, _L)

    def group(g, carry):
        base = g * _L
        acc = jnp.zeros((_L,), jnp.float32)
        for i in range(_L):
            b = base + i
            u0 = urows[b, pl.ds(0, _L)]
            u1 = urows[b, pl.ds(_L, _L)]
            p0 = prows[b, pl.ds(0, _L)]
            p1 = prows[b, pl.ds(_L, _L)]
            tot = jnp.sum(u0 * p0 + u1 * p1)
            acc = jnp.where(lanes == i, tot, acc)
        outc[pl.ds(base, _L)] = 1.0 / (1.0 + jnp.exp(-acc))
        return carry

    lax.fori_loop(0, _BPW // _L, group, 0)
    pltpu.sync_copy(outc, out_hbm.at[pl.ds(wid * _BPW, _BPW)])


@jax.jit
def kernel(user_ids, post_ids, user_table, post_table):
    uid = user_ids.astype(jnp.int32).reshape(_B // _CHUNK, _CHUNK)
    pid = post_ids.astype(jnp.int32).reshape(_B // _CHUNK, _CHUNK)
    mesh = plsc.VectorSubcoreMesh(core_axis_name="c", subcore_axis_name="s")

    tp = pl.kernel(
        _tp_body,
        out_type=(jax.ShapeDtypeStruct((_NCOL * 128 * _D,), jnp.float32),
                  jax.ShapeDtypeStruct((_NCOL * 128 * _D,), jnp.float32)),
        mesh=mesh,
        compiler_params=pltpu.CompilerParams(
            needs_layout_passes=False, disable_bounds_checks=True),
        scratch_types=[
            pltpu.VMEM((_D, 128), jnp.float32),
            pltpu.VMEM((_D, 128), jnp.float32),
            pltpu.VMEM((128 * _D,), jnp.float32),
            pltpu.VMEM((128 * _D,), jnp.float32),
            pltpu.SemaphoreType.DMA,
            pltpu.SemaphoreType.DMA,
            pltpu.SemaphoreType.DMA,
            pltpu.SemaphoreType.DMA,
        ],
    )
    ulin_flat, plin_flat = tp(user_table.T, post_table.T)
    ulin = ulin_flat.reshape(_NCOL * 128, _D)
    plin = plin_flat.reshape(_NCOL * 128, _D)

    f = pl.kernel(
        _cf_body,
        out_type=jax.ShapeDtypeStruct((_B,), jnp.float32),
        mesh=mesh,
        compiler_params=pltpu.CompilerParams(
            needs_layout_passes=False, use_tc_tiling_on_sc=False),
        scratch_types=[
            pltpu.VMEM((_NCHUNK, _CHUNK), jnp.int32),
            pltpu.VMEM((_NCHUNK, _CHUNK), jnp.int32),
            pltpu.VMEM((_BPW, _D), jnp.float32),
            pltpu.VMEM((_BPW, _D), jnp.float32),
            pltpu.VMEM((_BPW,), jnp.float32),
            pltpu.SemaphoreType.DMA,
            pltpu.SemaphoreType.DMA,
        ],
    )
    return f(uid, pid, ulin, plin)
